# baseline (speedup 1.0000x reference)
import functools

import jax
import jax.numpy as jnp
from jax.experimental import pallas as pl
from jax.experimental.pallas import tpu as pltpu

N = 1000000
M = 1000
K = 16
B = 8000
G = N // B


def _body(x_ref, wb_ref, u_ref, xt_ref, wbt_ref, tV_ref, bV_ref, sc_ref,
          out_ref, acc_u, acc_s):
    i = pl.program_id(0)
    xb = x_ref[0]
    ub = u_ref[0]
    pu = jnp.dot(xb, ub, preferred_element_type=jnp.float32)
    pb = jnp.sum(xb * wb_ref[0])

    @pl.when(i == 0)
    def _init():
        acc_u[...] = pu
        acc_s[...] = pb.reshape(1, 1)

    @pl.when(i > 0)
    def _acc():
        acc_u[...] = acc_u[...] + pu
        acc_s[...] = acc_s[...] + pb.reshape(1, 1)

    @pl.when(i == G - 1)
    def _final():
        xt = xt_ref[...]
        tmask = xt[:, :M]
        bmask = xt[:, M:]
        tV = tV_ref[...]
        bV = bV_ref[...]
        t_vec = jnp.dot(tmask, tV, preferred_element_type=jnp.float32)
        b_sum = jnp.dot(bmask, bV, preferred_element_type=jnp.float32)
        rowsq = jnp.sum(bV * bV, axis=1, keepdims=True)
        sq = jnp.dot(bmask, rowsq, preferred_element_type=jnp.float32)
        bias = acc_s[...][0, 0] + jnp.sum(xt * wbt_ref[...])
        u_vec = acc_u[...]
        u_t = jnp.sum(u_vec * t_vec)
        t_b = jnp.sum(t_vec * b_sum)
        u_b = jnp.sum(u_vec * b_sum)
        bs = 0.5 * (jnp.sum(b_sum * b_sum) - sq[0, 0])
        scv = sc_ref[...]
        w0 = scv[0, 0]
        delta = scv[0, 1]
        y = w0 + bias + u_t + t_b + bs + u_b
        z = y * delta
        a = -z
        res = jnp.maximum(a, 0.0) + jnp.log1p(jnp.exp(-jnp.abs(a)))
        out_ref[...] = res.reshape(1, 1)


@jax.jit
def kernel(x, delta, pmi, w_0, w_bias, u_V, t_V, b_V):
    del pmi
    x2 = x[:N].reshape(G, 1, B)
    wb = w_bias.reshape(-1)
    wb2 = wb[:N].reshape(G, 1, B)
    xt = x[N:].reshape(1, 2 * M)
    wbt = wb[N:].reshape(1, 2 * M)
    u3 = u_V.reshape(G, B, K)
    sc = jnp.concatenate([w_0, delta]).reshape(1, 2)

    out = pl.pallas_call(
        _body,
        grid=(G,),
        in_specs=[
            pl.BlockSpec((1, 1, B), lambda i: (i, 0, 0)),
            pl.BlockSpec((1, 1, B), lambda i: (i, 0, 0)),
            pl.BlockSpec((1, B, K), lambda i: (i, 0, 0)),
            pl.BlockSpec((1, 2 * M), lambda i: (0, 0)),
            pl.BlockSpec((1, 2 * M), lambda i: (0, 0)),
            pl.BlockSpec((M, K), lambda i: (0, 0)),
            pl.BlockSpec((M, K), lambda i: (0, 0)),
            pl.BlockSpec((1, 2), lambda i: (0, 0)),
        ],
        out_specs=pl.BlockSpec((1, 1), lambda i: (0, 0)),
        out_shape=jax.ShapeDtypeStruct((1, 1), jnp.float32),
        scratch_shapes=[
            pltpu.VMEM((1, K), jnp.float32),
            pltpu.VMEM((1, 1), jnp.float32),
        ],
    )(x2, wb2, u3, xt, wbt, t_V, b_V, sc)
    return out
